# baseline (speedup 1.0000x reference)
import functools
import math

import jax
import jax.numpy as jnp
from jax import lax
from jax.experimental import pallas as pl
from jax.experimental.pallas import tpu as pltpu
from jax.experimental.pallas import tpu_sc as plsc

_D = 128
_SCALE = math.sqrt(float(_D))


def _scale_body(x_ref, o_ref):
    o_ref[...] = x_ref[...] * _SCALE


@functools.lru_cache(maxsize=None)
def _make_scale(V, D):
    blk = 2000
    assert V % blk == 0
    return pl.pallas_call(
        _scale_body,
        out_shape=jax.ShapeDtypeStruct((V, D), jnp.float32),
        grid=(V // blk,),
        in_specs=[pl.BlockSpec((blk, D), lambda i: (i, 0))],
        out_specs=pl.BlockSpec((blk, D), lambda i: (i, 0)),
    )


@functools.lru_cache(maxsize=None)
def _make_gather(V, D, B):
    info = plsc.get_sparse_core_info()
    NC, NS = info.num_cores, info.num_subcores
    NW = NC * NS
    C = 128
    G = 1
    CB = C * G
    NB = 6
    assert B % (NW * CB) == 0
    b_per_w = B // NW
    n_idx = b_per_w // C
    n_chunks = b_per_w // CB
    mesh = plsc.VectorSubcoreMesh(core_axis_name="c", subcore_axis_name="s")

    @functools.partial(
        pl.kernel,
        out_type=jax.ShapeDtypeStruct((B, D), jnp.float32),
        mesh=mesh,
        scratch_types=[
            pltpu.VMEM((n_idx, C), jnp.int32),
            pltpu.VMEM((CB, D), jnp.float32),
            pltpu.VMEM((CB, D), jnp.float32),
            pltpu.VMEM((CB, D), jnp.float32),
            pltpu.VMEM((CB, D), jnp.float32),
            pltpu.VMEM((CB, D), jnp.float32),
            pltpu.VMEM((CB, D), jnp.float32),
            pltpu.SemaphoreType.DMA,
            pltpu.SemaphoreType.DMA,
            pltpu.SemaphoreType.DMA,
            pltpu.SemaphoreType.DMA,
            pltpu.SemaphoreType.DMA,
            pltpu.SemaphoreType.DMA,
            pltpu.SemaphoreType.DMA,
            pltpu.SemaphoreType.DMA,
            pltpu.SemaphoreType.DMA,
            pltpu.SemaphoreType.DMA,
            pltpu.SemaphoreType.DMA,
            pltpu.SemaphoreType.DMA,
        ],
    )
    def gather_kernel(idx_hbm, table_hbm, out_hbm,
                      idx_v, rows0, rows1, rows2, rows3, rows4, rows5,
                      g0, g1, g2, g3, g4, g5, s0, s1, s2, s3, s4, s5):
        wid = lax.axis_index("s") * NC + lax.axis_index("c")
        base = wid * b_per_w
        rows = (rows0, rows1, rows2, rows3, rows4, rows5)
        gsem = (g0, g1, g2, g3, g4, g5)
        ssem = (s0, s1, s2, s3, s4, s5)

        pltpu.sync_copy(idx_hbm.at[pl.ds(wid * n_idx, n_idx)], idx_v)

        def gather_start(i, b):
            for g in range(G):
                pltpu.async_copy(table_hbm.at[idx_v.at[i * G + g]],
                                 rows[b].at[pl.ds(g * C, C)], gsem[b])

        def gather_wait(i, b):
            for g in range(G):
                pltpu.make_async_copy(
                    table_hbm.at[idx_v.at[i * G + g]],
                    rows[b].at[pl.ds(g * C, C)], gsem[b]).wait()

        def scale_buf(b):
            def sbody(r, carry):
                for u in range(2):
                    for k in range(D // 16):
                        sl = (2 * r + u, pl.ds(16 * k, 16))
                        rows[b][sl] = rows[b][sl] * _SCALE
                return carry
            lax.fori_loop(0, CB // 2, sbody, 0)

        def scatter_start(i, b):
            pltpu.async_copy(
                rows[b], out_hbm.at[pl.ds(base + i * CB, CB)], ssem[b])

        def scatter_wait(i, b):
            pltpu.make_async_copy(
                rows[b], out_hbm.at[pl.ds(base + i * CB, CB)], ssem[b]).wait()

        def consume(i, b):
            gather_wait(i, b)
            scale_buf(b)
            scatter_start(i, b)

        gather_start(0, 0)
        gather_start(1, 1)
        gather_start(2, 2)
        consume(0, 0)
        gather_start(3, 3)
        consume(1, 1)
        gather_start(4, 4)
        consume(2, 2)
        gather_start(5, 5)

        def body(jj, carry):
            i = 6 * jj
            for o in (3, 4, 5, 6, 7, 8):
                b = o % 6
                b3 = (o + 3) % 6
                scatter_wait(i + o - 3, b3)
                gather_start(i + o + 3, b3)
                consume(i + o, b)
            return carry

        lax.fori_loop(0, (n_chunks - 8) // 6, body, 0)

        scatter_wait(n_chunks - 8, 0)
        gather_start(n_chunks - 2, 0)
        scatter_wait(n_chunks - 7, 1)
        gather_start(n_chunks - 1, 1)
        consume(n_chunks - 5, 3)
        consume(n_chunks - 4, 4)
        consume(n_chunks - 3, 5)
        consume(n_chunks - 2, 0)
        consume(n_chunks - 1, 1)
        for i in range(6):
            scatter_wait(n_chunks - 6 + i, (n_chunks - 6 + i) % 6)

    return gather_kernel


def kernel(Tokens, table):
    S, T = Tokens.shape
    V, D = table.shape
    B = S * T
    idx2d = Tokens.reshape(B // 128, 128).astype(jnp.int32)
    out = _make_gather(V, D, B)(idx2d, table)
    return out.reshape(S, T, D)


# speedup vs baseline: 1.0005x; 1.0005x over previous
import functools
import math

import jax
import jax.numpy as jnp
from jax import lax
from jax.experimental import pallas as pl
from jax.experimental.pallas import tpu as pltpu
from jax.experimental.pallas import tpu_sc as plsc

_D = 128
_SCALE = math.sqrt(float(_D))


@functools.lru_cache(maxsize=None)
def _make_gather(V, D, B):
    info = plsc.get_sparse_core_info()
    NC, NS = info.num_cores, info.num_subcores
    NW = NC * NS
    C = 128
    G = 1
    CB = C * G
    NB = 6
    assert B % (NW * CB) == 0
    b_per_w = B // NW
    n_idx = b_per_w // C
    n_chunks = b_per_w // CB
    mesh = plsc.VectorSubcoreMesh(core_axis_name="c", subcore_axis_name="s")

    @functools.partial(
        pl.kernel,
        out_type=jax.ShapeDtypeStruct((B, D), jnp.float32),
        mesh=mesh,
        scratch_types=[
            pltpu.VMEM((n_idx, C), jnp.int32),
            pltpu.VMEM((CB, D), jnp.float32),
            pltpu.VMEM((CB, D), jnp.float32),
            pltpu.VMEM((CB, D), jnp.float32),
            pltpu.VMEM((CB, D), jnp.float32),
            pltpu.VMEM((CB, D), jnp.float32),
            pltpu.VMEM((CB, D), jnp.float32),
            pltpu.SemaphoreType.DMA,
            pltpu.SemaphoreType.DMA,
            pltpu.SemaphoreType.DMA,
            pltpu.SemaphoreType.DMA,
            pltpu.SemaphoreType.DMA,
            pltpu.SemaphoreType.DMA,
            pltpu.SemaphoreType.DMA,
            pltpu.SemaphoreType.DMA,
            pltpu.SemaphoreType.DMA,
            pltpu.SemaphoreType.DMA,
            pltpu.SemaphoreType.DMA,
            pltpu.SemaphoreType.DMA,
        ],
    )
    def gather_kernel(idx_hbm, table_hbm, out_hbm,
                      idx_v, rows0, rows1, rows2, rows3, rows4, rows5,
                      g0, g1, g2, g3, g4, g5, s0, s1, s2, s3, s4, s5):
        wid = lax.axis_index("s") * NC + lax.axis_index("c")
        base = wid * b_per_w
        rows = (rows0, rows1, rows2, rows3, rows4, rows5)
        gsem = (g0, g1, g2, g3, g4, g5)
        ssem = (s0, s1, s2, s3, s4, s5)

        pltpu.sync_copy(idx_hbm.at[pl.ds(wid * n_idx, n_idx)], idx_v)

        def gather_start(i, b):
            for g in range(G):
                pltpu.async_copy(table_hbm.at[idx_v.at[i * G + g]],
                                 rows[b].at[pl.ds(g * C, C)], gsem[b])

        def gather_wait(i, b):
            for g in range(G):
                pltpu.make_async_copy(
                    table_hbm.at[idx_v.at[i * G + g]],
                    rows[b].at[pl.ds(g * C, C)], gsem[b]).wait()

        def scale_buf(b):
            def sbody(r, carry):
                for u in range(2):
                    for k in range(D // 16):
                        sl = (2 * r + u, pl.ds(16 * k, 16))
                        rows[b][sl] = rows[b][sl] * _SCALE
                return carry
            lax.fori_loop(0, CB // 2, sbody, 0)

        def scatter_start(i, b):
            pltpu.async_copy(
                rows[b], out_hbm.at[pl.ds(base + i * CB, CB)], ssem[b])

        def scatter_wait(i, b):
            pltpu.make_async_copy(
                rows[b], out_hbm.at[pl.ds(base + i * CB, CB)], ssem[b]).wait()

        def consume(i, b):
            gather_wait(i, b)
            scale_buf(b)
            scatter_start(i, b)

        gather_start(0, 0)
        gather_start(1, 1)
        gather_start(2, 2)
        consume(0, 0)
        gather_start(3, 3)
        consume(1, 1)
        gather_start(4, 4)
        consume(2, 2)
        gather_start(5, 5)

        def body(jj, carry):
            i = 6 * jj
            for o in (3, 4, 5, 6, 7, 8):
                b = o % 6
                b3 = (o + 3) % 6
                scatter_wait(i + o - 3, b3)
                gather_start(i + o + 3, b3)
                consume(i + o, b)
            return carry

        lax.fori_loop(0, (n_chunks - 8) // 6, body, 0)

        scatter_wait(n_chunks - 8, 0)
        gather_start(n_chunks - 2, 0)
        scatter_wait(n_chunks - 7, 1)
        gather_start(n_chunks - 1, 1)
        consume(n_chunks - 5, 3)
        consume(n_chunks - 4, 4)
        consume(n_chunks - 3, 5)
        consume(n_chunks - 2, 0)
        consume(n_chunks - 1, 1)
        for i in range(6):
            scatter_wait(n_chunks - 6 + i, (n_chunks - 6 + i) % 6)

    return gather_kernel


def kernel(Tokens, table):
    S, T = Tokens.shape
    V, D = table.shape
    B = S * T
    idx2d = Tokens.reshape(B // 128, 128).astype(jnp.int32)
    out = _make_gather(V, D, B)(idx2d, table)
    return out.reshape(S, T, D)
